# baseline (speedup 1.0000x reference)
import jax
import jax.numpy as jnp
from jax.experimental import pallas as pl
from jax.experimental.pallas import tpu as pltpu

_BLOCK_COLS = 49152
_BLOCK_COLS_I = 8192


def _make_body(nblk_i2):
    def _copy_body(u_ref, i_ref, uo_ref, io_ref):
        uo_ref[...] = u_ref[...]

        @pl.when(pl.program_id(0) < nblk_i2)
        def _():
            io_ref[...] = i_ref[...]

    return _copy_body


def kernel(embed_user, embed_item):
    ut = embed_user.T
    it = embed_item.T
    dim, ucols = ut.shape
    icols = it.shape[1]
    half = dim // 2
    nblk_u = (ucols + _BLOCK_COLS - 1) // _BLOCK_COLS
    nblk_i = (icols + _BLOCK_COLS_I - 1) // _BLOCK_COLS_I

    u_spec = pl.BlockSpec((half, _BLOCK_COLS), lambda j: (j % 2, j // 2))
    i_spec = pl.BlockSpec(
        (half, _BLOCK_COLS_I),
        lambda j: (j % 2, jnp.minimum(j // 2, nblk_i - 1)))
    out_u, out_i = pl.pallas_call(
        _make_body(2 * nblk_i),
        grid=(2 * max(nblk_u, nblk_i),),
        in_specs=[u_spec, i_spec],
        out_specs=[u_spec, i_spec],
        out_shape=[
            jax.ShapeDtypeStruct(ut.shape, ut.dtype),
            jax.ShapeDtypeStruct(it.shape, it.dtype),
        ],
        compiler_params=pltpu.CompilerParams(
            dimension_semantics=("arbitrary",),
        ),
    )(ut, it)
    return (out_u.T, out_i.T)


# speedup vs baseline: 1.0266x; 1.0266x over previous
import jax
import jax.numpy as jnp
from jax.experimental import pallas as pl
from jax.experimental.pallas import tpu as pltpu

_BLOCK_COLS = 45056
_BLOCK_COLS_I = 8192


def _make_body(nblk_i):
    def _copy_body(u_ref, i_ref, uo_ref, io_ref):
        uo_ref[...] = u_ref[...]

        @pl.when(pl.program_id(0) < nblk_i)
        def _():
            io_ref[...] = i_ref[...]

    return _copy_body


def kernel(embed_user, embed_item):
    ut = embed_user.T
    it = embed_item.T
    dim, ucols = ut.shape
    icols = it.shape[1]
    nblk_u = (ucols + _BLOCK_COLS - 1) // _BLOCK_COLS
    nblk_i = (icols + _BLOCK_COLS_I - 1) // _BLOCK_COLS_I

    u_spec = pl.BlockSpec((dim, _BLOCK_COLS), lambda j: (0, j))
    i_spec = pl.BlockSpec((dim, _BLOCK_COLS_I),
                          lambda j: (0, jnp.minimum(j, nblk_i - 1)))
    out_u, out_i = pl.pallas_call(
        _make_body(nblk_i),
        grid=(max(nblk_u, nblk_i),),
        in_specs=[u_spec, i_spec],
        out_specs=[u_spec, i_spec],
        out_shape=[
            jax.ShapeDtypeStruct(ut.shape, ut.dtype),
            jax.ShapeDtypeStruct(it.shape, it.dtype),
        ],
        compiler_params=pltpu.CompilerParams(
            dimension_semantics=("arbitrary",),
        ),
    )(ut, it)
    return (out_u.T, out_i.T)


# speedup vs baseline: 1.0268x; 1.0002x over previous
import jax
import jax.numpy as jnp
from jax.experimental import pallas as pl
from jax.experimental.pallas import tpu as pltpu

_BLOCK_COLS = 49152
_BLOCK_COLS_I = 8192


def _make_body(nblk_i):
    def _copy_body(u_ref, i_ref, uo_ref, io_ref):
        uo_ref[...] = u_ref[...]

        @pl.when(pl.program_id(0) < nblk_i)
        def _():
            io_ref[...] = i_ref[...]

    return _copy_body


def kernel(embed_user, embed_item):
    ut = embed_user.T
    it = embed_item.T
    dim, ucols = ut.shape
    icols = it.shape[1]
    nblk_u = (ucols + _BLOCK_COLS - 1) // _BLOCK_COLS
    nblk_i = (icols + _BLOCK_COLS_I - 1) // _BLOCK_COLS_I

    u_spec = pl.BlockSpec((dim, _BLOCK_COLS), lambda j: (0, j))
    i_spec = pl.BlockSpec((dim, _BLOCK_COLS_I),
                          lambda j: (0, jnp.minimum(j, nblk_i - 1)))
    out_u, out_i = pl.pallas_call(
        _make_body(nblk_i),
        grid=(max(nblk_u, nblk_i),),
        in_specs=[u_spec, i_spec],
        out_specs=[u_spec, i_spec],
        out_shape=[
            jax.ShapeDtypeStruct(ut.shape, ut.dtype),
            jax.ShapeDtypeStruct(it.shape, it.dtype),
        ],
        compiler_params=pltpu.CompilerParams(
            dimension_semantics=("arbitrary",),
        ),
    )(ut, it)
    return (out_u.T, out_i.T)
